# Initial kernel scaffold; baseline (speedup 1.0000x reference)
import jax
import jax.numpy as jnp
from jax.experimental import pallas as pl


def kernel(user_feat_0, user_feat_1, item_feat_0, item_feat_1, user_ids, item_ids, A_row, A_col, A_val, user_table, item_table, uW0, uW1, iW0, iW1, fc1_W, fc1_b, fc2_W, fc2_b, out_W, out_b):
    raise NotImplementedError("write your pallas kernel here")


# speedup vs baseline: 12.8160x; 12.8160x over previous
import dataclasses
import functools

import jax
import jax.numpy as jnp
from jax import lax
from jax.experimental import pallas as pl
from jax.experimental.pallas import tpu as pltpu
from jax.experimental.pallas import tpu_sc as plsc

N_USERS = 100000
N_ITEMS = 100000
N = N_USERS + N_ITEMS
HALF = N // 2
EMB = 16
NNZ = 3200000
B = 4096
GCN_LAYERS = 3

NC = 2
NS = 16
L = 16
W = 128

NNZ_PAD = ((NNZ + NS * W - 1) // (NS * W)) * (NS * W)
EDGES_PER_TILE = NNZ_PAD // NS
N_WIN = EDGES_PER_TILE // W

RPT = 6256
RPT_LAST = HALF - (NS - 1) * RPT
ZCH = 784
GARB = 8192
GARB_BASE = HALF
ACC_PAD = max((NS - 1) * RPT + 8 * ZCH, GARB_BASE + GARB)

_mesh = plsc.VectorSubcoreMesh(core_axis_name="c", subcore_axis_name="s")

_cp = pltpu.CompilerParams(needs_layout_passes=False,
                           use_tc_tiling_on_sc=False)


def _sc_layer(emb_in, a_row, a_col, a_val):
    @functools.partial(
        pl.kernel,
        out_type=jax.ShapeDtypeStruct((N, EMB), jnp.float32),
        mesh=_mesh,
        compiler_params=_cp,
        scratch_types=[
            pltpu.VMEM_SHARED((ACC_PAD, EMB), jnp.float32),
            pltpu.VMEM((W,), jnp.int32),
            pltpu.VMEM((W,), jnp.int32),
            pltpu.VMEM((W,), jnp.float32),
            pltpu.VMEM((W,), jnp.int32),
            pltpu.VMEM((W, EMB), jnp.float32),
            pltpu.VMEM((ZCH, EMB), jnp.float32),
        ],
    )
    def k(emb_hbm, row_hbm, col_hbm, val_hbm, out_hbm,
          acc, colv, rowv, valv, sidxv, rows, zbuf):
        c = lax.axis_index("c")
        s = lax.axis_index("s")
        base_row = c * HALF
        iota16 = lax.iota(jnp.int32, L)

        zero16 = jnp.zeros((L,), jnp.float32)

        for r in range(ZCH):
            zbuf[r] = zero16

        acc_base = s * RPT

        for i in range(8):
            pltpu.sync_copy(zbuf, acc.at[pl.ds(acc_base + i * ZCH, ZCH)])
        plsc.subcore_barrier()

        tile_base = s * EDGES_PER_TILE

        @pl.loop(0, N_WIN)
        def _(w):
            base = tile_base + w * W
            pltpu.sync_copy(col_hbm.at[pl.ds(base, W)], colv)
            pltpu.sync_copy(row_hbm.at[pl.ds(base, W)], rowv)
            pltpu.sync_copy(val_hbm.at[pl.ds(base, W)], valv)
            pltpu.sync_copy(emb_hbm.at[colv], rows)
            for j in range(0, W, L):
                rv = rowv[pl.ds(j, L)]
                lidx = rv - base_row
                inb = (lidx >= 0) & (lidx < HALF)
                fb = GARB_BASE + ((iota16 + (base + j)) & (GARB - 1))
                sidx = jnp.where(inb, lidx, fb)
                vv = valv[pl.ds(j, L)]
                for t in range(L):
                    e = j + t
                    rows[e] = rows[e] * vv[t]
                pltpu.sync_copy(rows.at[pl.ds(j, L)], acc.at[sidx], add=True)

        plsc.subcore_barrier()

        @pl.when(s < NS - 1)
        def _():
            pltpu.sync_copy(acc.at[pl.ds(acc_base, RPT)],
                            out_hbm.at[pl.ds(base_row + acc_base, RPT)])

        @pl.when(s == NS - 1)
        def _():
            pltpu.sync_copy(acc.at[pl.ds(acc_base, RPT_LAST)],
                            out_hbm.at[pl.ds(base_row + acc_base, RPT_LAST)])

    return k(emb_in, a_row, a_col, a_val)


def _sc_lookup(e0, e1, e2, e3, user_ids, item_ids):
    per_tile = B // (NC * NS)

    @functools.partial(
        pl.kernel,
        out_type=[jax.ShapeDtypeStruct((B, EMB), jnp.float32),
                  jax.ShapeDtypeStruct((B, EMB), jnp.float32)],
        mesh=_mesh,
        compiler_params=_cp,
        scratch_types=[
            pltpu.VMEM((per_tile,), jnp.int32),
            pltpu.VMEM((per_tile,), jnp.int32),
            pltpu.VMEM((per_tile, EMB), jnp.float32),
            pltpu.VMEM((per_tile, EMB), jnp.float32),
            pltpu.VMEM((per_tile, EMB), jnp.float32),
            pltpu.VMEM((per_tile, EMB), jnp.float32),
            pltpu.VMEM((per_tile, EMB), jnp.float32),
        ],
    )
    def k(e0_hbm, e1_hbm, e2_hbm, e3_hbm, uid_hbm, iid_hbm, xu_hbm, xi_hbm,
          idv, idv2, g0, g1, g2, g3, xbuf):
        c = lax.axis_index("c")
        s = lax.axis_index("s")
        wid = s * NC + c
        b0 = wid * per_tile

        def avg4():
            for r in range(per_tile):
                xbuf[r] = (g0[r] + g1[r] + g2[r] + g3[r]) * 0.25

        pltpu.sync_copy(uid_hbm.at[pl.ds(b0, per_tile)], idv)
        pltpu.sync_copy(e0_hbm.at[idv], g0)
        pltpu.sync_copy(e1_hbm.at[idv], g1)
        pltpu.sync_copy(e2_hbm.at[idv], g2)
        pltpu.sync_copy(e3_hbm.at[idv], g3)
        avg4()
        pltpu.sync_copy(xbuf, xu_hbm.at[pl.ds(b0, per_tile)])

        pltpu.sync_copy(iid_hbm.at[pl.ds(b0, per_tile)], idv)
        for j in range(0, per_tile, L):
            idv2[pl.ds(j, L)] = idv[pl.ds(j, L)] + N_USERS
        pltpu.sync_copy(e0_hbm.at[idv2], g0)
        pltpu.sync_copy(e1_hbm.at[idv2], g1)
        pltpu.sync_copy(e2_hbm.at[idv2], g2)
        pltpu.sync_copy(e3_hbm.at[idv2], g3)
        avg4()
        pltpu.sync_copy(xbuf, xi_hbm.at[pl.ds(b0, per_tile)])

    return k(e0, e1, e2, e3, user_ids, item_ids)


def _tc_mlp(xu, xi, w1u, w1i, b1, w2, b2, w3, b3):
    def body(xu_ref, xi_ref, w1u_ref, w1i_ref, b1_ref, w2_ref, b2_ref,
             w3_ref, b3_ref, o_ref):
        h = jnp.dot(xu_ref[...], w1u_ref[...], precision=lax.Precision.HIGHEST,
                    preferred_element_type=jnp.float32)
        h += jnp.dot(xi_ref[...], w1i_ref[...], precision=lax.Precision.HIGHEST,
                     preferred_element_type=jnp.float32)
        h = jnp.maximum(h + b1_ref[...], 0.0)
        h = jnp.maximum(
            jnp.dot(h, w2_ref[...], precision=lax.Precision.HIGHEST,
                    preferred_element_type=jnp.float32)
            + b2_ref[...], 0.0)
        o_ref[...] = (jnp.dot(h, w3_ref[...], precision=lax.Precision.HIGHEST,
                              preferred_element_type=jnp.float32)
                      + b3_ref[...])

    return pl.pallas_call(
        body,
        out_shape=jax.ShapeDtypeStruct((B, 1), jnp.float32),
    )(xu, xi, w1u, w1i, b1.reshape(1, -1), w2, b2.reshape(1, -1),
      w3, b3.reshape(1, -1))


def kernel(user_feat_0, user_feat_1, item_feat_0, item_feat_1, user_ids,
           item_ids, A_row, A_col, A_val, user_table, item_table, uW0, uW1,
           iW0, iW1, fc1_W, fc1_b, fc2_W, fc2_b, out_W, out_b):
    del user_feat_0, user_feat_1, item_feat_0, item_feat_1, uW0, uW1, iW0, iW1

    pad = NNZ_PAD - NNZ
    a_row = jnp.pad(A_row, (0, pad))
    a_col = jnp.pad(A_col, (0, pad))
    a_val = jnp.pad(A_val, (0, pad))

    e0 = jnp.concatenate([user_table, item_table], axis=0)
    e1 = _sc_layer(e0, a_row, a_col, a_val)
    e2 = _sc_layer(e1, a_row, a_col, a_val)
    e3 = _sc_layer(e2, a_row, a_col, a_val)

    xu, xi = _sc_lookup(e0, e1, e2, e3, user_ids, item_ids)

    return _tc_mlp(xu, xi, fc1_W[:EMB], fc1_W[EMB:], fc1_b, fc2_W, fc2_b,
                   out_W, out_b)


# speedup vs baseline: 44.0478x; 3.4369x over previous
import dataclasses
import functools

import jax
import jax.numpy as jnp
from jax import lax
from jax.experimental import pallas as pl
from jax.experimental.pallas import tpu as pltpu
from jax.experimental.pallas import tpu_sc as plsc

N_USERS = 100000
N_ITEMS = 100000
N = N_USERS + N_ITEMS
HALF = N // 2
EMB = 16
NNZ = 3200000
B = 4096
GCN_LAYERS = 3

NC = 2
NS = 16
L = 16
W = 128

NNZ_PAD = ((NNZ + NS * W * 2 - 1) // (NS * W * 2)) * (NS * W * 2)
EDGES_PER_TILE = NNZ_PAD // NS
N_WIN = EDGES_PER_TILE // W
PAIRS = N_WIN // 2

RPT = 6256
RPT_LAST = HALF - (NS - 1) * RPT
ZCH = 784
GARB = 8192
GARB_BASE = HALF
ACC_PAD = max((NS - 1) * RPT + 8 * ZCH, GARB_BASE + GARB)

_mesh = plsc.VectorSubcoreMesh(core_axis_name="c", subcore_axis_name="s")

_cp = pltpu.CompilerParams(needs_layout_passes=False,
                           use_tc_tiling_on_sc=False)


def _sc_layer(emb_in, a_row, a_col, a_val):
    @functools.partial(
        pl.kernel,
        out_type=jax.ShapeDtypeStruct((N, EMB), jnp.float32),
        mesh=_mesh,
        compiler_params=_cp,
        scratch_types=[
            pltpu.VMEM_SHARED((ACC_PAD, EMB), jnp.float32),
            pltpu.VMEM((W,), jnp.int32),
            pltpu.VMEM((W,), jnp.int32),
            pltpu.VMEM((W,), jnp.int32),
            pltpu.VMEM((W,), jnp.int32),
            pltpu.VMEM((W,), jnp.float32),
            pltpu.VMEM((W,), jnp.float32),
            pltpu.VMEM((W,), jnp.int32),
            pltpu.VMEM((W, EMB), jnp.float32),
            pltpu.VMEM((W, EMB), jnp.float32),
            pltpu.VMEM((ZCH, EMB), jnp.float32),
            pltpu.SemaphoreType.DMA,
            pltpu.SemaphoreType.DMA,
            pltpu.SemaphoreType.DMA,
            pltpu.SemaphoreType.DMA,
        ],
    )
    def k(emb_hbm, row_hbm, col_hbm, val_hbm, out_hbm,
          acc, cb0, cb1, rb0, rb1, vb0, vb1, sidxv, rows0, rows1, zbuf,
          lsem0, lsem1, gsem0, gsem1):
        c = lax.axis_index("c")
        s = lax.axis_index("s")
        base_row = c * HALF
        iota16 = lax.iota(jnp.int32, L)

        zero16 = jnp.zeros((L,), jnp.float32)

        for r in range(ZCH):
            zbuf[r] = zero16

        acc_base = s * RPT

        for i in range(8):
            pltpu.sync_copy(zbuf, acc.at[pl.ds(acc_base + i * ZCH, ZCH)])
        plsc.subcore_barrier()

        tile_base = s * EDGES_PER_TILE
        bufs = ((cb0, rb0, vb0, rows0, lsem0, gsem0),
                (cb1, rb1, vb1, rows1, lsem1, gsem1))

        def issue_lin(w, b):
            cb, rb, vb, _, ls, _ = bufs[b]
            base = tile_base + w * W
            pltpu.async_copy(col_hbm.at[pl.ds(base, W)], cb, ls)
            pltpu.async_copy(row_hbm.at[pl.ds(base, W)], rb, ls)
            pltpu.async_copy(val_hbm.at[pl.ds(base, W)], vb, ls)

        def wait_lin(b):
            cb, rb, vb, _, ls, _ = bufs[b]
            pltpu.make_async_copy(col_hbm.at[pl.ds(0, W)], cb, ls).wait()
            pltpu.make_async_copy(row_hbm.at[pl.ds(0, W)], rb, ls).wait()
            pltpu.make_async_copy(val_hbm.at[pl.ds(0, W)], vb, ls).wait()

        def issue_gather(b):
            cb, _, _, rw, _, gs = bufs[b]
            pltpu.async_copy(emb_hbm.at[cb], rw, gs)

        def wait_gather(b):
            cb, _, _, rw, _, gs = bufs[b]
            pltpu.make_async_copy(emb_hbm.at[cb], rw, gs).wait()

        def compute_scatter(w, b):
            _, rb, vb, rw, _, _ = bufs[b]
            base = tile_base + w * W
            for j in range(0, W, L):
                rv = rb[pl.ds(j, L)]
                lidx = rv - base_row
                inb = (lidx >= 0) & (lidx < HALF)
                fb = GARB_BASE + ((iota16 + (base + j)) & (GARB - 1))
                sidxv[pl.ds(j, L)] = jnp.where(inb, lidx, fb)
                vv = vb[pl.ds(j, L)]
                for t in range(L):
                    e = j + t
                    rw[e] = rw[e] * vv[t]
            pltpu.sync_copy(rw, acc.at[sidxv], add=True)

        issue_lin(0, 0)
        wait_lin(0)
        issue_gather(0)
        issue_lin(1, 1)

        @pl.loop(0, PAIRS - 1)
        def _(g):
            w0 = g * 2
            for b in (0, 1):
                w = w0 + b
                wait_lin(1 - b)
                issue_gather(1 - b)
                wait_gather(b)
                compute_scatter(w, b)
                issue_lin(w + 2, b)

        w0 = (PAIRS - 1) * 2
        wait_lin(1)
        issue_gather(1)
        wait_gather(0)
        compute_scatter(w0, 0)
        wait_gather(1)
        compute_scatter(w0 + 1, 1)

        plsc.subcore_barrier()

        @pl.when(s < NS - 1)
        def _():
            pltpu.sync_copy(acc.at[pl.ds(acc_base, RPT)],
                            out_hbm.at[pl.ds(base_row + acc_base, RPT)])

        @pl.when(s == NS - 1)
        def _():
            pltpu.sync_copy(acc.at[pl.ds(acc_base, RPT_LAST)],
                            out_hbm.at[pl.ds(base_row + acc_base, RPT_LAST)])

    return k(emb_in, a_row, a_col, a_val)


def _sc_lookup(e0, e1, e2, e3, user_ids, item_ids):
    per_tile = B // (NC * NS)

    @functools.partial(
        pl.kernel,
        out_type=[jax.ShapeDtypeStruct((B, EMB), jnp.float32),
                  jax.ShapeDtypeStruct((B, EMB), jnp.float32)],
        mesh=_mesh,
        compiler_params=_cp,
        scratch_types=[
            pltpu.VMEM((per_tile,), jnp.int32),
            pltpu.VMEM((per_tile,), jnp.int32),
            pltpu.VMEM((per_tile, EMB), jnp.float32),
            pltpu.VMEM((per_tile, EMB), jnp.float32),
            pltpu.VMEM((per_tile, EMB), jnp.float32),
            pltpu.VMEM((per_tile, EMB), jnp.float32),
            pltpu.VMEM((per_tile, EMB), jnp.float32),
        ],
    )
    def k(e0_hbm, e1_hbm, e2_hbm, e3_hbm, uid_hbm, iid_hbm, xu_hbm, xi_hbm,
          idv, idv2, g0, g1, g2, g3, xbuf):
        c = lax.axis_index("c")
        s = lax.axis_index("s")
        wid = s * NC + c
        b0 = wid * per_tile

        def avg4():
            for r in range(per_tile):
                xbuf[r] = (g0[r] + g1[r] + g2[r] + g3[r]) * 0.25

        pltpu.sync_copy(uid_hbm.at[pl.ds(b0, per_tile)], idv)
        pltpu.sync_copy(e0_hbm.at[idv], g0)
        pltpu.sync_copy(e1_hbm.at[idv], g1)
        pltpu.sync_copy(e2_hbm.at[idv], g2)
        pltpu.sync_copy(e3_hbm.at[idv], g3)
        avg4()
        pltpu.sync_copy(xbuf, xu_hbm.at[pl.ds(b0, per_tile)])

        pltpu.sync_copy(iid_hbm.at[pl.ds(b0, per_tile)], idv)
        for j in range(0, per_tile, L):
            idv2[pl.ds(j, L)] = idv[pl.ds(j, L)] + N_USERS
        pltpu.sync_copy(e0_hbm.at[idv2], g0)
        pltpu.sync_copy(e1_hbm.at[idv2], g1)
        pltpu.sync_copy(e2_hbm.at[idv2], g2)
        pltpu.sync_copy(e3_hbm.at[idv2], g3)
        avg4()
        pltpu.sync_copy(xbuf, xi_hbm.at[pl.ds(b0, per_tile)])

    return k(e0, e1, e2, e3, user_ids, item_ids)


def _tc_mlp(xu, xi, w1u, w1i, b1, w2, b2, w3, b3):
    def body(xu_ref, xi_ref, w1u_ref, w1i_ref, b1_ref, w2_ref, b2_ref,
             w3_ref, b3_ref, o_ref):
        h = jnp.dot(xu_ref[...], w1u_ref[...], precision=lax.Precision.HIGHEST,
                    preferred_element_type=jnp.float32)
        h += jnp.dot(xi_ref[...], w1i_ref[...], precision=lax.Precision.HIGHEST,
                     preferred_element_type=jnp.float32)
        h = jnp.maximum(h + b1_ref[...], 0.0)
        h = jnp.maximum(
            jnp.dot(h, w2_ref[...], precision=lax.Precision.HIGHEST,
                    preferred_element_type=jnp.float32)
            + b2_ref[...], 0.0)
        o_ref[...] = (jnp.dot(h, w3_ref[...], precision=lax.Precision.HIGHEST,
                              preferred_element_type=jnp.float32)
                      + b3_ref[...])

    return pl.pallas_call(
        body,
        out_shape=jax.ShapeDtypeStruct((B, 1), jnp.float32),
    )(xu, xi, w1u, w1i, b1.reshape(1, -1), w2, b2.reshape(1, -1),
      w3, b3.reshape(1, -1))


def kernel(user_feat_0, user_feat_1, item_feat_0, item_feat_1, user_ids,
           item_ids, A_row, A_col, A_val, user_table, item_table, uW0, uW1,
           iW0, iW1, fc1_W, fc1_b, fc2_W, fc2_b, out_W, out_b):
    del user_feat_0, user_feat_1, item_feat_0, item_feat_1, uW0, uW1, iW0, iW1

    pad = NNZ_PAD - NNZ
    a_row = jnp.pad(A_row, (0, pad))
    a_col = jnp.pad(A_col, (0, pad))
    a_val = jnp.pad(A_val, (0, pad))

    e0 = jnp.concatenate([user_table, item_table], axis=0)
    e1 = _sc_layer(e0, a_row, a_col, a_val)
    e2 = _sc_layer(e1, a_row, a_col, a_val)
    e3 = _sc_layer(e2, a_row, a_col, a_val)

    xu, xi = _sc_lookup(e0, e1, e2, e3, user_ids, item_ids)

    return _tc_mlp(xu, xi, fc1_W[:EMB], fc1_W[EMB:], fc1_b, fc2_W, fc2_b,
                   out_W, out_b)


# speedup vs baseline: 68.2318x; 1.5490x over previous
import dataclasses
import functools

import jax
import jax.numpy as jnp
from jax import lax
from jax.experimental import pallas as pl
from jax.experimental.pallas import tpu as pltpu
from jax.experimental.pallas import tpu_sc as plsc

N_USERS = 100000
N_ITEMS = 100000
N = N_USERS + N_ITEMS
HALF = N // 2
EMB = 16
NNZ = 3200000
B = 4096
GCN_LAYERS = 3

NC = 2
NS = 16
L = 16
W = 128

SW = 4
SUPER_E = SW * W

_CHUNK = NS * SUPER_E * 2
NNZ_PAD = ((NNZ + _CHUNK - 1) // _CHUNK) * _CHUNK
EDGES_PER_TILE = NNZ_PAD // NS
N_WIN = EDGES_PER_TILE // W
NSUP = N_WIN // SW
HPAIRS = NSUP // 2

RPT = 6256
RPT_LAST = HALF - (NS - 1) * RPT
ZCH = 112
GARB = 8192
GARB_BASE = HALF
ZREP = 56
ACC_PAD = max((NS - 1) * RPT + ZREP * ZCH, GARB_BASE + GARB)

_mesh = plsc.VectorSubcoreMesh(core_axis_name="c", subcore_axis_name="s")

_cp = pltpu.CompilerParams(needs_layout_passes=False,
                           use_tc_tiling_on_sc=False)


def _sc_layer(emb_in, a_row, a_col, a_val):
    @functools.partial(
        pl.kernel,
        out_type=jax.ShapeDtypeStruct((N, EMB), jnp.float32),
        mesh=_mesh,
        compiler_params=_cp,
        scratch_types=[
            pltpu.VMEM_SHARED((ACC_PAD, EMB), jnp.float32),
            pltpu.VMEM((SUPER_E,), jnp.int32),
            pltpu.VMEM((SUPER_E,), jnp.int32),
            pltpu.VMEM((SUPER_E,), jnp.int32),
            pltpu.VMEM((SUPER_E,), jnp.int32),
            pltpu.VMEM((SUPER_E,), jnp.float32),
            pltpu.VMEM((SUPER_E,), jnp.float32),
            pltpu.VMEM((W,), jnp.int32),
            pltpu.VMEM((W,), jnp.int32),
            pltpu.VMEM((W, EMB), jnp.float32),
            pltpu.VMEM((W, EMB), jnp.float32),
            pltpu.VMEM((W, EMB), jnp.float32),
            pltpu.VMEM((W, EMB), jnp.float32),
            pltpu.VMEM((ZCH, EMB), jnp.float32),
            pltpu.SemaphoreType.DMA,
            pltpu.SemaphoreType.DMA,
            pltpu.SemaphoreType.DMA,
            pltpu.SemaphoreType.DMA,
            pltpu.SemaphoreType.DMA,
            pltpu.SemaphoreType.DMA,
        ],
    )
    def k(emb_hbm, row_hbm, col_hbm, val_hbm, out_hbm,
          acc, cbA, cbB, rbA, rbB, vbA, vbB, si0, si1, rw0, rw1, sb0, sb1,
          zbuf, lsA, lsB, gs0, gs1, ss0, ss1):
        c = lax.axis_index("c")
        s = lax.axis_index("s")
        base_row = c * HALF
        iota16 = lax.iota(jnp.int32, L)

        zero16 = jnp.zeros((L,), jnp.float32)
        for r in range(ZCH):
            zbuf[r] = zero16

        acc_base = s * RPT

        @pl.loop(0, ZREP)
        def _(i):
            pltpu.sync_copy(zbuf, acc.at[pl.ds(acc_base + i * ZCH, ZCH)])
        plsc.subcore_barrier()

        tile_base = s * EDGES_PER_TILE
        lin = ((cbA, rbA, vbA, lsA), (cbB, rbB, vbB, lsB))
        gat = ((rw0, gs0), (rw1, gs1))
        sca = ((sb0, si0, ss0), (sb1, si1, ss1))

        def issue_lin(sup, S):
            cb, rb, vb, ls = lin[S]
            base = tile_base + sup * SUPER_E
            pltpu.async_copy(col_hbm.at[pl.ds(base, SUPER_E)], cb, ls)
            pltpu.async_copy(row_hbm.at[pl.ds(base, SUPER_E)], rb, ls)
            pltpu.async_copy(val_hbm.at[pl.ds(base, SUPER_E)], vb, ls)

        def wait_lin(S):
            cb, rb, vb, ls = lin[S]
            pltpu.make_async_copy(col_hbm.at[pl.ds(0, SUPER_E)], cb, ls).wait()
            pltpu.make_async_copy(row_hbm.at[pl.ds(0, SUPER_E)], rb, ls).wait()
            pltpu.make_async_copy(val_hbm.at[pl.ds(0, SUPER_E)], vb, ls).wait()

        def issue_gather(S, k, p):
            rw, gs = gat[p]
            idx = lin[S][0].at[pl.ds(k * W, W)]
            pltpu.async_copy(emb_hbm.at[idx], rw, gs)

        def wait_gather(p):
            rw, gs = gat[p]
            pltpu.make_async_copy(emb_hbm.at[cbA.at[pl.ds(0, W)]],
                                  rw, gs).wait()

        def issue_scatter(p):
            sb, si, ss = sca[p]
            pltpu.async_copy(sb, acc.at[si], ss, add=True)

        def wait_scatter(p):
            sb, si, ss = sca[p]
            pltpu.make_async_copy(sb, acc.at[si], ss).wait()

        def compute(g, S, k, p):
            _, rb, vb, _ = lin[S]
            rw, _ = gat[p]
            sb, si, _ = sca[p]
            base = tile_base + g * SUPER_E + k * W
            for j in range(0, W, L):
                o = k * W + j
                rv = rb[pl.ds(o, L)]
                lidx = rv - base_row
                inb = (lidx >= 0) & (lidx < HALF)
                fb = GARB_BASE + ((iota16 + (base + j)) & (GARB - 1))
                si[pl.ds(j, L)] = jnp.where(inb, lidx, fb)
                vv = vb[pl.ds(o, L)]
                for t in range(L):
                    sb[j + t] = rw[j + t] * vv[t]

        issue_lin(0, 0)
        wait_lin(0)
        issue_gather(0, 0, 0)
        issue_lin(1, 1)

        @pl.loop(0, HPAIRS)
        def _(h):
            for S in (0, 1):
                g = h * 2 + S
                for k in range(SW):
                    p = k & 1
                    if k == SW - 1:
                        def _pref():
                            wait_lin(1 - S)
                            issue_gather(1 - S, 0, 1 - p)
                        if S == 0:
                            _pref()
                        else:
                            pl.when(h < HPAIRS - 1)(_pref)
                    else:
                        issue_gather(S, k + 1, 1 - p)
                    wait_gather(p)
                    if S == 0 and k < 2:
                        pl.when(h > 0)(lambda pp=p: wait_scatter(pp))
                    else:
                        wait_scatter(p)
                    compute(g, S, k, p)
                    issue_scatter(p)
                    if k == SW - 1:
                        pl.when(h < HPAIRS - 1)(lambda SS=S: issue_lin(
                            h * 2 + SS + 2, SS))

        wait_scatter(0)
        wait_scatter(1)
        plsc.subcore_barrier()

        @pl.when(s < NS - 1)
        def _():
            pltpu.sync_copy(acc.at[pl.ds(acc_base, RPT)],
                            out_hbm.at[pl.ds(base_row + acc_base, RPT)])

        @pl.when(s == NS - 1)
        def _():
            pltpu.sync_copy(acc.at[pl.ds(acc_base, RPT_LAST)],
                            out_hbm.at[pl.ds(base_row + acc_base, RPT_LAST)])

    return k(emb_in, a_row, a_col, a_val)


def _sc_lookup(e0, e1, e2, e3, user_ids, item_ids):
    per_tile = B // (NC * NS)

    @functools.partial(
        pl.kernel,
        out_type=[jax.ShapeDtypeStruct((B, EMB), jnp.float32),
                  jax.ShapeDtypeStruct((B, EMB), jnp.float32)],
        mesh=_mesh,
        compiler_params=_cp,
        scratch_types=[
            pltpu.VMEM((per_tile,), jnp.int32),
            pltpu.VMEM((per_tile,), jnp.int32),
            pltpu.VMEM((per_tile, EMB), jnp.float32),
            pltpu.VMEM((per_tile, EMB), jnp.float32),
            pltpu.VMEM((per_tile, EMB), jnp.float32),
            pltpu.VMEM((per_tile, EMB), jnp.float32),
            pltpu.VMEM((per_tile, EMB), jnp.float32),
        ],
    )
    def k(e0_hbm, e1_hbm, e2_hbm, e3_hbm, uid_hbm, iid_hbm, xu_hbm, xi_hbm,
          idv, idv2, g0, g1, g2, g3, xbuf):
        c = lax.axis_index("c")
        s = lax.axis_index("s")
        wid = s * NC + c
        b0 = wid * per_tile

        def avg4():
            for r in range(per_tile):
                xbuf[r] = (g0[r] + g1[r] + g2[r] + g3[r]) * 0.25

        pltpu.sync_copy(uid_hbm.at[pl.ds(b0, per_tile)], idv)
        pltpu.sync_copy(e0_hbm.at[idv], g0)
        pltpu.sync_copy(e1_hbm.at[idv], g1)
        pltpu.sync_copy(e2_hbm.at[idv], g2)
        pltpu.sync_copy(e3_hbm.at[idv], g3)
        avg4()
        pltpu.sync_copy(xbuf, xu_hbm.at[pl.ds(b0, per_tile)])

        pltpu.sync_copy(iid_hbm.at[pl.ds(b0, per_tile)], idv)
        for j in range(0, per_tile, L):
            idv2[pl.ds(j, L)] = idv[pl.ds(j, L)] + N_USERS
        pltpu.sync_copy(e0_hbm.at[idv2], g0)
        pltpu.sync_copy(e1_hbm.at[idv2], g1)
        pltpu.sync_copy(e2_hbm.at[idv2], g2)
        pltpu.sync_copy(e3_hbm.at[idv2], g3)
        avg4()
        pltpu.sync_copy(xbuf, xi_hbm.at[pl.ds(b0, per_tile)])

    return k(e0, e1, e2, e3, user_ids, item_ids)


def _tc_mlp(xu, xi, w1u, w1i, b1, w2, b2, w3, b3):
    def body(xu_ref, xi_ref, w1u_ref, w1i_ref, b1_ref, w2_ref, b2_ref,
             w3_ref, b3_ref, o_ref):
        h = jnp.dot(xu_ref[...], w1u_ref[...], precision=lax.Precision.HIGHEST,
                    preferred_element_type=jnp.float32)
        h += jnp.dot(xi_ref[...], w1i_ref[...], precision=lax.Precision.HIGHEST,
                     preferred_element_type=jnp.float32)
        h = jnp.maximum(h + b1_ref[...], 0.0)
        h = jnp.maximum(
            jnp.dot(h, w2_ref[...], precision=lax.Precision.HIGHEST,
                    preferred_element_type=jnp.float32)
            + b2_ref[...], 0.0)
        o_ref[...] = (jnp.dot(h, w3_ref[...], precision=lax.Precision.HIGHEST,
                              preferred_element_type=jnp.float32)
                      + b3_ref[...])

    return pl.pallas_call(
        body,
        out_shape=jax.ShapeDtypeStruct((B, 1), jnp.float32),
    )(xu, xi, w1u, w1i, b1.reshape(1, -1), w2, b2.reshape(1, -1),
      w3, b3.reshape(1, -1))


def kernel(user_feat_0, user_feat_1, item_feat_0, item_feat_1, user_ids,
           item_ids, A_row, A_col, A_val, user_table, item_table, uW0, uW1,
           iW0, iW1, fc1_W, fc1_b, fc2_W, fc2_b, out_W, out_b):
    del user_feat_0, user_feat_1, item_feat_0, item_feat_1, uW0, uW1, iW0, iW1

    pad = NNZ_PAD - NNZ
    a_row = jnp.pad(A_row, (0, pad))
    a_col = jnp.pad(A_col, (0, pad))
    a_val = jnp.pad(A_val, (0, pad))

    e0 = jnp.concatenate([user_table, item_table], axis=0)
    e1 = _sc_layer(e0, a_row, a_col, a_val)
    e2 = _sc_layer(e1, a_row, a_col, a_val)
    e3 = _sc_layer(e2, a_row, a_col, a_val)

    xu, xi = _sc_lookup(e0, e1, e2, e3, user_ids, item_ids)

    return _tc_mlp(xu, xi, fc1_W[:EMB], fc1_W[EMB:], fc1_b, fc2_W, fc2_b,
                   out_W, out_b)


# speedup vs baseline: 99.3971x; 1.4568x over previous
import dataclasses
import functools

import jax
import jax.numpy as jnp
from jax import lax
from jax.experimental import pallas as pl
from jax.experimental.pallas import tpu as pltpu
from jax.experimental.pallas import tpu_sc as plsc

N_USERS = 100000
N_ITEMS = 100000
N = N_USERS + N_ITEMS
HALF = N // 2
EMB = 16
NNZ = 3200000
B = 4096
GCN_LAYERS = 3

NC = 2
NS = 16
L = 16
W = 128

SW = 4
SUPER_E = SW * W

_CHUNK = NS * SUPER_E * 2
NNZ_PAD = ((NNZ + _CHUNK - 1) // _CHUNK) * _CHUNK
EDGES_PER_TILE = NNZ_PAD // NS
N_WIN = EDGES_PER_TILE // W
NSUP = N_WIN // SW
HPAIRS = NSUP // 2

RPT = 6256
RPT_LAST = HALF - (NS - 1) * RPT
ZCH = 112
GARB = 8192
GARB_BASE = HALF
ZREP = 56
ACC_PAD = max((NS - 1) * RPT + ZREP * ZCH, GARB_BASE + GARB)

_mesh = plsc.VectorSubcoreMesh(core_axis_name="c", subcore_axis_name="s")

_cp = pltpu.CompilerParams(needs_layout_passes=False,
                           use_tc_tiling_on_sc=False)


def _sc_layer(emb_in, a_row, a_col, a_val):
    @functools.partial(
        pl.kernel,
        out_type=jax.ShapeDtypeStruct((N, EMB), jnp.float32),
        mesh=_mesh,
        compiler_params=_cp,
        scratch_types=[
            pltpu.VMEM_SHARED((ACC_PAD, EMB), jnp.float32),
            pltpu.VMEM((SUPER_E,), jnp.int32),
            pltpu.VMEM((SUPER_E,), jnp.int32),
            pltpu.VMEM((SUPER_E,), jnp.int32),
            pltpu.VMEM((SUPER_E,), jnp.int32),
            pltpu.VMEM((SUPER_E,), jnp.float32),
            pltpu.VMEM((SUPER_E,), jnp.float32),
        ] + [pltpu.VMEM((W,), jnp.int32)] * SW
          + [pltpu.VMEM((W, EMB), jnp.float32)] * SW
          + [pltpu.VMEM((W, EMB), jnp.float32)] * SW
          + [pltpu.VMEM((ZCH, EMB), jnp.float32)]
          + [pltpu.SemaphoreType.DMA] * 2
          + [pltpu.SemaphoreType.DMA] * SW
          + [pltpu.SemaphoreType.DMA] * SW,
    )
    def k(emb_hbm, row_hbm, col_hbm, val_hbm, out_hbm,
          acc, cbA, cbB, rbA, rbB, vbA, vbB,
          si0, si1, si2, si3, rw0, rw1, rw2, rw3, sb0, sb1, sb2, sb3,
          zbuf, lsA, lsB, gs0, gs1, gs2, gs3, ss0, ss1, ss2, ss3):
        c = lax.axis_index("c")
        s = lax.axis_index("s")
        base_row = c * HALF
        iota16 = lax.iota(jnp.int32, L)

        zero16 = jnp.zeros((L,), jnp.float32)
        for r in range(ZCH):
            zbuf[r] = zero16

        acc_base = s * RPT

        @pl.loop(0, ZREP)
        def _(i):
            pltpu.sync_copy(zbuf, acc.at[pl.ds(acc_base + i * ZCH, ZCH)])
        plsc.subcore_barrier()

        tile_base = s * EDGES_PER_TILE
        lin = ((cbA, rbA, vbA, lsA), (cbB, rbB, vbB, lsB))
        gat = ((rw0, gs0), (rw1, gs1), (rw2, gs2), (rw3, gs3))
        sca = ((sb0, si0, ss0), (sb1, si1, ss1), (sb2, si2, ss2),
               (sb3, si3, ss3))

        def issue_lin(sup, S):
            cb, rb, vb, ls = lin[S]
            base = tile_base + sup * SUPER_E
            pltpu.async_copy(col_hbm.at[pl.ds(base, SUPER_E)], cb, ls)
            pltpu.async_copy(row_hbm.at[pl.ds(base, SUPER_E)], rb, ls)
            pltpu.async_copy(val_hbm.at[pl.ds(base, SUPER_E)], vb, ls)

        def wait_lin(S):
            cb, rb, vb, ls = lin[S]
            pltpu.make_async_copy(col_hbm.at[pl.ds(0, SUPER_E)], cb, ls).wait()
            pltpu.make_async_copy(row_hbm.at[pl.ds(0, SUPER_E)], rb, ls).wait()
            pltpu.make_async_copy(val_hbm.at[pl.ds(0, SUPER_E)], vb, ls).wait()

        def issue_gather(S, k, p):
            rw, gs = gat[p]
            idx = lin[S][0].at[pl.ds(k * W, W)]
            pltpu.async_copy(emb_hbm.at[idx], rw, gs)

        def wait_gather(p):
            rw, gs = gat[p]
            pltpu.make_async_copy(emb_hbm.at[cbA.at[pl.ds(0, W)]],
                                  rw, gs).wait()

        def issue_scatter(p):
            sb, si, ss = sca[p]
            pltpu.async_copy(sb, acc.at[si], ss, add=True)

        def wait_scatter(p):
            sb, si, ss = sca[p]
            pltpu.make_async_copy(sb, acc.at[si], ss).wait()

        def compute(g, S, k, p):
            _, rb, vb, _ = lin[S]
            rw, _ = gat[p]
            sb, si, _ = sca[p]
            base = tile_base + g * SUPER_E + k * W
            for j in range(0, W, L):
                o = k * W + j
                rv = rb[pl.ds(o, L)]
                lidx = rv - base_row
                inb = (lidx >= 0) & (lidx < HALF)
                fb = GARB_BASE + ((iota16 + (base + j)) & (GARB - 1))
                si[pl.ds(j, L)] = jnp.where(inb, lidx, fb)
                vv = vb[pl.ds(o, L)]
                for t in range(L):
                    sb[j + t] = rw[j + t] * vv[t]

        issue_lin(0, 0)
        wait_lin(0)
        issue_gather(0, 0, 0)
        issue_gather(0, 1, 1)
        issue_gather(0, 2, 2)
        issue_lin(1, 1)

        @pl.loop(0, HPAIRS)
        def _(h):
            for S in (0, 1):
                g = h * 2 + S
                for k in range(SW):
                    if k == 1:
                        if S == 0:
                            wait_lin(1)
                        else:
                            pl.when(h < HPAIRS - 1)(lambda: wait_lin(0))
                    if k == 0:
                        issue_gather(S, 3, 3)
                    else:
                        def _pref(SS=S, kk=k):
                            issue_gather(1 - SS, kk - 1, kk - 1)
                        if S == 0:
                            _pref()
                        else:
                            pl.when(h < HPAIRS - 1)(_pref)
                    wait_gather(k)
                    if S == 0:
                        pl.when(h > 0)(lambda kk=k: wait_scatter(kk))
                    else:
                        wait_scatter(k)
                    compute(g, S, k, k)
                    issue_scatter(k)
                    if k == SW - 1:
                        pl.when(h < HPAIRS - 1)(lambda SS=S: issue_lin(
                            h * 2 + SS + 2, SS))

        for q in range(SW):
            wait_scatter(q)
        plsc.subcore_barrier()

        @pl.when(s < NS - 1)
        def _():
            pltpu.sync_copy(acc.at[pl.ds(acc_base, RPT)],
                            out_hbm.at[pl.ds(base_row + acc_base, RPT)])

        @pl.when(s == NS - 1)
        def _():
            pltpu.sync_copy(acc.at[pl.ds(acc_base, RPT_LAST)],
                            out_hbm.at[pl.ds(base_row + acc_base, RPT_LAST)])

    return k(emb_in, a_row, a_col, a_val)


def _sc_lookup(e0, e1, e2, e3, user_ids, item_ids):
    per_tile = B // (NC * NS)

    @functools.partial(
        pl.kernel,
        out_type=[jax.ShapeDtypeStruct((B, EMB), jnp.float32),
                  jax.ShapeDtypeStruct((B, EMB), jnp.float32)],
        mesh=_mesh,
        compiler_params=_cp,
        scratch_types=[
            pltpu.VMEM((per_tile,), jnp.int32),
            pltpu.VMEM((per_tile,), jnp.int32),
            pltpu.VMEM((per_tile, EMB), jnp.float32),
            pltpu.VMEM((per_tile, EMB), jnp.float32),
            pltpu.VMEM((per_tile, EMB), jnp.float32),
            pltpu.VMEM((per_tile, EMB), jnp.float32),
            pltpu.VMEM((per_tile, EMB), jnp.float32),
        ],
    )
    def k(e0_hbm, e1_hbm, e2_hbm, e3_hbm, uid_hbm, iid_hbm, xu_hbm, xi_hbm,
          idv, idv2, g0, g1, g2, g3, xbuf):
        c = lax.axis_index("c")
        s = lax.axis_index("s")
        wid = s * NC + c
        b0 = wid * per_tile

        def avg4():
            for r in range(per_tile):
                xbuf[r] = (g0[r] + g1[r] + g2[r] + g3[r]) * 0.25

        pltpu.sync_copy(uid_hbm.at[pl.ds(b0, per_tile)], idv)
        pltpu.sync_copy(e0_hbm.at[idv], g0)
        pltpu.sync_copy(e1_hbm.at[idv], g1)
        pltpu.sync_copy(e2_hbm.at[idv], g2)
        pltpu.sync_copy(e3_hbm.at[idv], g3)
        avg4()
        pltpu.sync_copy(xbuf, xu_hbm.at[pl.ds(b0, per_tile)])

        pltpu.sync_copy(iid_hbm.at[pl.ds(b0, per_tile)], idv)
        for j in range(0, per_tile, L):
            idv2[pl.ds(j, L)] = idv[pl.ds(j, L)] + N_USERS
        pltpu.sync_copy(e0_hbm.at[idv2], g0)
        pltpu.sync_copy(e1_hbm.at[idv2], g1)
        pltpu.sync_copy(e2_hbm.at[idv2], g2)
        pltpu.sync_copy(e3_hbm.at[idv2], g3)
        avg4()
        pltpu.sync_copy(xbuf, xi_hbm.at[pl.ds(b0, per_tile)])

    return k(e0, e1, e2, e3, user_ids, item_ids)


def _tc_mlp(xu, xi, w1u, w1i, b1, w2, b2, w3, b3):
    def body(xu_ref, xi_ref, w1u_ref, w1i_ref, b1_ref, w2_ref, b2_ref,
             w3_ref, b3_ref, o_ref):
        h = jnp.dot(xu_ref[...], w1u_ref[...], precision=lax.Precision.HIGHEST,
                    preferred_element_type=jnp.float32)
        h += jnp.dot(xi_ref[...], w1i_ref[...], precision=lax.Precision.HIGHEST,
                     preferred_element_type=jnp.float32)
        h = jnp.maximum(h + b1_ref[...], 0.0)
        h = jnp.maximum(
            jnp.dot(h, w2_ref[...], precision=lax.Precision.HIGHEST,
                    preferred_element_type=jnp.float32)
            + b2_ref[...], 0.0)
        o_ref[...] = (jnp.dot(h, w3_ref[...], precision=lax.Precision.HIGHEST,
                              preferred_element_type=jnp.float32)
                      + b3_ref[...])

    return pl.pallas_call(
        body,
        out_shape=jax.ShapeDtypeStruct((B, 1), jnp.float32),
    )(xu, xi, w1u, w1i, b1.reshape(1, -1), w2, b2.reshape(1, -1),
      w3, b3.reshape(1, -1))


def kernel(user_feat_0, user_feat_1, item_feat_0, item_feat_1, user_ids,
           item_ids, A_row, A_col, A_val, user_table, item_table, uW0, uW1,
           iW0, iW1, fc1_W, fc1_b, fc2_W, fc2_b, out_W, out_b):
    del user_feat_0, user_feat_1, item_feat_0, item_feat_1, uW0, uW1, iW0, iW1

    pad = NNZ_PAD - NNZ
    a_row = jnp.pad(A_row, (0, pad))
    a_col = jnp.pad(A_col, (0, pad))
    a_val = jnp.pad(A_val, (0, pad))

    e0 = jnp.concatenate([user_table, item_table], axis=0)
    e1 = _sc_layer(e0, a_row, a_col, a_val)
    e2 = _sc_layer(e1, a_row, a_col, a_val)
    e3 = _sc_layer(e2, a_row, a_col, a_val)

    xu, xi = _sc_lookup(e0, e1, e2, e3, user_ids, item_ids)

    return _tc_mlp(xu, xi, fc1_W[:EMB], fc1_W[EMB:], fc1_b, fc2_W, fc2_b,
                   out_W, out_b)
